# baseline (speedup 1.0000x reference)
import jax
import jax.numpy as jnp
from jax.experimental import pallas as pl
from jax.experimental.pallas import tpu as pltpu

_BATCH = 16384
_INPUT_DIM = 1000
_EMBED_DIM = 16
_BM = 2048


def _mm_body(x_ref, e_ref, o_ref):
    o_ref[...] = jnp.dot(x_ref[...], e_ref[...],
                         preferred_element_type=jnp.float32)


def kernel(x, embedding):
    grid = (_BATCH // _BM,)
    return pl.pallas_call(
        _mm_body,
        grid=grid,
        in_specs=[
            pl.BlockSpec((_BM, _INPUT_DIM), lambda i: (i, 0)),
            pl.BlockSpec((_INPUT_DIM, _EMBED_DIM), lambda i: (0, 0)),
        ],
        out_specs=pl.BlockSpec((_BM, _EMBED_DIM), lambda i: (i, 0)),
        out_shape=jax.ShapeDtypeStruct((_BATCH, _EMBED_DIM), jnp.float32),
        compiler_params=pltpu.CompilerParams(
            dimension_semantics=("arbitrary",),
        ),
    )(x, embedding)
